# baseline (speedup 1.0000x reference)
import functools

import jax
import jax.numpy as jnp
from jax import lax
from jax.experimental import pallas as pl
from jax.experimental.pallas import tpu as pltpu
from jax.experimental.pallas import tpu_sc as plsc

_NC = 2
_NS = 16
_NW = _NC * _NS

_B = 16384
_H = 200
_D = 16
_HT = _H // 8
_BT = _B // 128
_UNIT = 8 * 128
_UNITS = _HT * _BT
_UPW = _UNITS // _NW
_PITCH = 129


def _make_kernel(v: int):
    out_words = _B * _H * _D

    mesh = plsc.VectorSubcoreMesh(core_axis_name="c", subcore_axis_name="s")

    @functools.partial(
        pl.kernel,
        out_type=jax.ShapeDtypeStruct((_H, 2, _BT, 8, 128), jnp.float32),
        mesh=mesh,
        compiler_params=pltpu.CompilerParams(
            use_tc_tiling_on_sc=False, needs_layout_passes=False),
        scratch_types=[
            pltpu.VMEM((_UNIT,), jnp.int32),
            pltpu.VMEM((_UNIT,), jnp.int32),
            pltpu.VMEM((_UNIT, _D), jnp.float32),
            pltpu.VMEM((_UNIT, _D), jnp.float32),
            pltpu.VMEM((8, _D, _PITCH), jnp.float32),
            pltpu.VMEM((8, _D, _PITCH), jnp.float32),
        ] + [pltpu.SemaphoreType.DMA] * 6,
    )
    def gather_kernel(idx_hbm, table_hbm, out_hbm,
                      idx0, idx1, rows0, rows1, t0, t1,
                      si0, si1, sg0, sg1, so0, so1):
        idx_v = (idx0, idx1)
        rows_v = (rows0, rows1)
        t_v = (t0, t1)
        si = (si0, si1)
        sg = (sg0, sg1)
        so = (so0, so1)

        wid = lax.axis_index("s") * _NC + lax.axis_index("c")
        u0 = wid * _UPW

        ev = lax.iota(jnp.int32, 16)

        def idx_off(u):
            return (u0 + u) * _UNIT

        def idx_load(u, p):
            pltpu.async_copy(
                idx_hbm.at[pl.ds(idx_off(u), _UNIT)], idx_v[p], si[p])

        def wait_idx(u, p):
            pltpu.make_async_copy(
                idx_hbm.at[pl.ds(idx_off(u), _UNIT)], idx_v[p], si[p]).wait()

        def gather(p):
            pltpu.async_copy(table_hbm.at[idx_v[p]], rows_v[p], sg[p])

        def wait_gather(p):
            pltpu.make_async_copy(table_hbm.at[idx_v[p]], rows_v[p], sg[p]).wait()

        def out_pairs(u, p):
            gu = u0 + u
            ht = gu // _BT
            bt = gu % _BT
            pairs = []
            for hs in range(8):
                for eb in range(2):
                    src = t_v[p].at[hs, pl.ds(eb * 8, 8), pl.ds(0, 128)]
                    dst = out_hbm.at[ht * 8 + hs, eb, bt]
                    pairs.append((src, dst))
            return pairs

        def outs(u, p):
            for src, dst in out_pairs(u, p):
                pltpu.async_copy(src, dst, so[p])

        def wait_outs(u, p):
            for src, dst in out_pairs(u, p):
                pltpu.make_async_copy(src, dst, so[p]).wait()

        def transpose(p):
            rows = rows_v[p]
            t = t_v[p]

            @plsc.parallel_loop(0, _UNIT, unroll=8)
            def _t(j):
                hs = lax.shift_right_logical(j, 7)
                bl = j & 127
                plsc.store_scatter(
                    t, [jnp.full((16,), hs, jnp.int32), ev,
                        jnp.full((16,), bl, jnp.int32)], rows[j])

        idx_load(0, 0)
        idx_load(1, 1)
        wait_idx(0, 0)
        gather(0)

        @pl.loop(0, _UPW, step=2)
        def _unit(ub):
            for p in range(2):
                u = ub + p
                q = 1 - p
                wait_gather(p)

                @pl.when(u + 1 < _UPW)
                def _():
                    wait_idx(u + 1, q)
                    gather(q)

                @pl.when(u + 2 < _UPW)
                def _():
                    idx_load(u + 2, p)

                @pl.when(u >= 2)
                def _():
                    wait_outs(u - 2, p)

                transpose(p)
                outs(u, p)

        wait_outs(_UPW - 2, 0)
        wait_outs(_UPW - 1, 1)

    return gather_kernel


def _make_tc_transpose(v: int):
    k = 7936
    grid = pl.cdiv(v, k)

    def body(i_ref, o_ref):
        blk = i_ref[...]
        blk3 = blk.reshape(16, k // 8, 8)
        o_ref[...] = blk3.transpose(1, 2, 0).reshape(k // 8, 128)

    return pl.pallas_call(
        body,
        grid=(grid,),
        in_specs=[pl.BlockSpec((16, k), lambda g: (0, g))],
        out_specs=pl.BlockSpec((k // 8, 128), lambda g: (g, 0)),
        out_shape=jax.ShapeDtypeStruct((v // 8, 128), jnp.float32),
    )


def kernel(farm_ids, table):
    b, h = farm_ids.shape
    v, d = table.shape
    assert (b, h, d) == (_B, _H, _D)
    idx_phys = (farm_ids.astype(jnp.int32)
                .reshape(_BT, 128, _HT, 8)
                .transpose(2, 0, 3, 1)
                .reshape(-1))
    table_rm = _make_tc_transpose(v)(table.T).reshape(v, d)
    out5 = _make_kernel(v)(idx_phys, table_rm)
    return out5.transpose(2, 4, 0, 1, 3).reshape(_B, _H, _D)


# speedup vs baseline: 2.3466x; 2.3466x over previous
import functools

import jax
import jax.numpy as jnp
from jax import lax
from jax.experimental import pallas as pl
from jax.experimental.pallas import tpu as pltpu
from jax.experimental.pallas import tpu_sc as plsc

_NC = 2
_NS = 16
_NW = _NC * _NS

_B = 16384
_H = 200
_D = 16
_HT = _H // 8
_BT = _B // 128
_UNIT = 8 * 128
_UNITS = _HT * _BT
_UPW = _UNITS // _NW
_PITCH = 129


def _make_kernel(v: int):
    out_words = _B * _H * _D

    mesh = plsc.VectorSubcoreMesh(core_axis_name="c", subcore_axis_name="s")

    @functools.partial(
        pl.kernel,
        out_type=jax.ShapeDtypeStruct((_H, 2, _BT, 8, 128), jnp.float32),
        mesh=mesh,
        compiler_params=pltpu.CompilerParams(
            use_tc_tiling_on_sc=False, needs_layout_passes=False),
        scratch_types=[
            pltpu.VMEM((_UNIT,), jnp.int32),
            pltpu.VMEM((_UNIT,), jnp.int32),
            pltpu.VMEM((_UNIT, _D), jnp.float32),
            pltpu.VMEM((_UNIT, _D), jnp.float32),
            pltpu.VMEM((8, _D, _PITCH), jnp.float32),
            pltpu.VMEM((8, _D, _PITCH), jnp.float32),
        ] + [pltpu.SemaphoreType.DMA] * 6,
    )
    def gather_kernel(idx_hbm, table_hbm, out_hbm,
                      idx0, idx1, rows0, rows1, t0, t1,
                      si0, si1, sg0, sg1, so0, so1):
        idx_v = (idx0, idx1)
        rows_v = (rows0, rows1)
        t_v = (t0, t1)
        si = (si0, si1)
        sg = (sg0, sg1)
        so = (so0, so1)

        wid = lax.axis_index("s") * _NC + lax.axis_index("c")
        u0 = wid * _UPW

        ev = lax.iota(jnp.int32, 16)

        def idx_off(u):
            return (u0 + u) * _UNIT

        def idx_load(u, p):
            pltpu.async_copy(
                idx_hbm.at[pl.ds(idx_off(u), _UNIT)], idx_v[p], si[p])

        def wait_idx(u, p):
            pltpu.make_async_copy(
                idx_hbm.at[pl.ds(idx_off(u), _UNIT)], idx_v[p], si[p]).wait()

        def scale_idx(p):
            ref = idx_v[p]

            @plsc.parallel_loop(0, _UNIT // 16, unroll=8)
            def _s(i):
                ref[pl.ds(i * 16, 16)] = ref[pl.ds(i * 16, 16)] * 8

        def gather(p):
            pltpu.async_copy(table_hbm.at[idx_v[p]], rows_v[p], sg[p])

        def wait_gather(p):
            pltpu.make_async_copy(table_hbm.at[idx_v[p]], rows_v[p], sg[p]).wait()

        def out_pairs(u, p):
            gu = u0 + u
            ht = gu // _BT
            bt = gu % _BT
            pairs = []
            for hs in range(8):
                for eb in range(2):
                    src = t_v[p].at[hs, pl.ds(eb * 8, 8), pl.ds(0, 128)]
                    dst = out_hbm.at[ht * 8 + hs, eb, bt]
                    pairs.append((src, dst))
            return pairs

        def outs(u, p):
            for src, dst in out_pairs(u, p):
                pltpu.async_copy(src, dst, so[p])

        def wait_outs(u, p):
            for src, dst in out_pairs(u, p):
                pltpu.make_async_copy(src, dst, so[p]).wait()

        def transpose(p):
            rows = rows_v[p]
            t = t_v[p]

            @plsc.parallel_loop(0, _UNIT, unroll=8)
            def _t(j):
                hs = lax.shift_right_logical(j, 7)
                bl = j & 127
                plsc.store_scatter(
                    t, [jnp.full((16,), hs, jnp.int32), ev,
                        jnp.full((16,), bl, jnp.int32)], rows[j])

        idx_load(0, 0)
        idx_load(1, 1)
        wait_idx(0, 0)
        scale_idx(0)
        gather(0)

        @pl.loop(0, _UPW, step=2)
        def _unit(ub):
            for p in range(2):
                u = ub + p
                q = 1 - p
                wait_gather(p)

                @pl.when(u + 1 < _UPW)
                def _():
                    wait_idx(u + 1, q)
                    scale_idx(q)
                    gather(q)

                @pl.when(u + 2 < _UPW)
                def _():
                    idx_load(u + 2, p)

                @pl.when(u >= 2)
                def _():
                    wait_outs(u - 2, p)

                transpose(p)
                outs(u, p)

        wait_outs(_UPW - 2, 0)
        wait_outs(_UPW - 1, 1)

    return gather_kernel


def _make_tc_transpose(v: int):
    k = 7936
    grid = pl.cdiv(v, k)

    def body(i_ref, o_ref):
        t = i_ref[...].T
        o_ref[...] = jnp.pad(t, ((0, 0), (0, 112)))

    return pl.pallas_call(
        body,
        grid=(grid,),
        in_specs=[pl.BlockSpec((16, k), lambda g: (0, g))],
        out_specs=pl.BlockSpec((k, 128), lambda g: (g, 0)),
        out_shape=jax.ShapeDtypeStruct((v, 128), jnp.float32),
    )


def kernel(farm_ids, table):
    b, h = farm_ids.shape
    v, d = table.shape
    assert (b, h, d) == (_B, _H, _D)
    idx_phys = (farm_ids.astype(jnp.int32)
                .reshape(_BT, 128, _HT, 8)
                .transpose(2, 0, 3, 1)
                .reshape(-1))
    table_rm = _make_tc_transpose(v)(table.T).reshape(v * 8, d)
    out5 = _make_kernel(v)(idx_phys, table_rm)
    return out5.transpose(2, 4, 0, 1, 3).reshape(_B, _H, _D)


# speedup vs baseline: 2.4815x; 1.0575x over previous
import functools

import jax
import jax.numpy as jnp
from jax import lax
from jax.experimental import pallas as pl
from jax.experimental.pallas import tpu as pltpu
from jax.experimental.pallas import tpu_sc as plsc

_NC = 2
_NS = 16
_NW = _NC * _NS

_B = 16384
_H = 200
_D = 16
_HT = _H // 8
_BT = _B // 128
_UNIT = 8 * 128
_UNITS = _HT * _BT
_UPW = _UNITS // _NW
_PITCH = 129


def _make_kernel(v: int):
    out_words = _B * _H * _D

    mesh = plsc.VectorSubcoreMesh(core_axis_name="c", subcore_axis_name="s")

    @functools.partial(
        pl.kernel,
        out_type=jax.ShapeDtypeStruct((_H, 2, _BT, 8, 128), jnp.float32),
        mesh=mesh,
        compiler_params=pltpu.CompilerParams(
            use_tc_tiling_on_sc=False, needs_layout_passes=False),
        scratch_types=[
            pltpu.VMEM((_UNIT,), jnp.int32),
            pltpu.VMEM((_UNIT,), jnp.int32),
            pltpu.VMEM((_UNIT, _D), jnp.float32),
            pltpu.VMEM((_UNIT, _D), jnp.float32),
            pltpu.VMEM((8, _D, _PITCH), jnp.float32),
            pltpu.VMEM((8, _D, _PITCH), jnp.float32),
        ] + [pltpu.SemaphoreType.DMA] * 8,
    )
    def gather_kernel(idx_hbm, table_hbm, out_hbm,
                      idx0, idx1, rows0, rows1, t0, t1,
                      si0, si1, sg0, sg1, sh0, sh1, so0, so1):
        idx_v = (idx0, idx1)
        rows_v = (rows0, rows1)
        t_v = (t0, t1)
        si = (si0, si1)
        sg = (sg0, sg1)
        sh = (sh0, sh1)
        so = (so0, so1)

        wid = lax.axis_index("s") * _NC + lax.axis_index("c")
        u0 = wid * _UPW

        ev = lax.iota(jnp.int32, 16)

        def idx_off(u):
            return (u0 + u) * _UNIT

        def idx_load(u, p):
            pltpu.async_copy(
                idx_hbm.at[pl.ds(idx_off(u), _UNIT)], idx_v[p], si[p])

        def wait_idx(u, p):
            pltpu.make_async_copy(
                idx_hbm.at[pl.ds(idx_off(u), _UNIT)], idx_v[p], si[p]).wait()

        def scale_idx(p):
            ref = idx_v[p]

            @plsc.parallel_loop(0, _UNIT // 16, unroll=8)
            def _s(i):
                ref[pl.ds(i * 16, 16)] = ref[pl.ds(i * 16, 16)] * 8

        _HU = _UNIT // 2

        def gather(p):
            pltpu.async_copy(table_hbm.at[idx_v[p].at[pl.ds(0, _HU)]],
                             rows_v[p].at[pl.ds(0, _HU)], sg[p])
            pltpu.async_copy(table_hbm.at[idx_v[p].at[pl.ds(_HU, _HU)]],
                             rows_v[p].at[pl.ds(_HU, _HU)], sh[p])

        def wait_gather(p):
            pltpu.make_async_copy(table_hbm.at[idx_v[p].at[pl.ds(0, _HU)]],
                                  rows_v[p].at[pl.ds(0, _HU)], sg[p]).wait()
            pltpu.make_async_copy(table_hbm.at[idx_v[p].at[pl.ds(_HU, _HU)]],
                                  rows_v[p].at[pl.ds(_HU, _HU)], sh[p]).wait()

        def out_pairs(u, p):
            gu = u0 + u
            ht = gu // _BT
            bt = gu % _BT
            pairs = []
            for hs in range(8):
                for eb in range(2):
                    src = t_v[p].at[hs, pl.ds(eb * 8, 8), pl.ds(0, 128)]
                    dst = out_hbm.at[ht * 8 + hs, eb, bt]
                    pairs.append((src, dst))
            return pairs

        def outs(u, p):
            for src, dst in out_pairs(u, p):
                pltpu.async_copy(src, dst, so[p])

        def wait_outs(u, p):
            for src, dst in out_pairs(u, p):
                pltpu.make_async_copy(src, dst, so[p]).wait()

        def transpose(p):
            rows = rows_v[p]
            t = t_v[p]

            @plsc.parallel_loop(0, _UNIT, unroll=8)
            def _t(j):
                hs = lax.shift_right_logical(j, 7)
                bl = j & 127
                plsc.store_scatter(
                    t, [jnp.full((16,), hs, jnp.int32), ev,
                        jnp.full((16,), bl, jnp.int32)], rows[j])

        idx_load(0, 0)
        idx_load(1, 1)
        wait_idx(0, 0)
        scale_idx(0)
        gather(0)

        @pl.loop(0, _UPW, step=2)
        def _unit(ub):
            for p in range(2):
                u = ub + p
                q = 1 - p
                wait_gather(p)

                @pl.when(u + 1 < _UPW)
                def _():
                    wait_idx(u + 1, q)
                    scale_idx(q)
                    gather(q)

                @pl.when(u + 2 < _UPW)
                def _():
                    idx_load(u + 2, p)

                @pl.when(u >= 2)
                def _():
                    wait_outs(u - 2, p)

                transpose(p)
                outs(u, p)

        wait_outs(_UPW - 2, 0)
        wait_outs(_UPW - 1, 1)

    return gather_kernel


def _make_tc_transpose(v: int):
    k = 7936
    grid = pl.cdiv(v, k)

    def body(i_ref, o_ref):
        t = i_ref[...].T
        o_ref[...] = jnp.pad(t, ((0, 0), (0, 112)))

    return pl.pallas_call(
        body,
        grid=(grid,),
        in_specs=[pl.BlockSpec((16, k), lambda g: (0, g))],
        out_specs=pl.BlockSpec((k, 128), lambda g: (g, 0)),
        out_shape=jax.ShapeDtypeStruct((v, 128), jnp.float32),
    )


def kernel(farm_ids, table):
    b, h = farm_ids.shape
    v, d = table.shape
    assert (b, h, d) == (_B, _H, _D)
    idx_phys = (farm_ids.astype(jnp.int32)
                .reshape(_BT, 128, _HT, 8)
                .transpose(2, 0, 3, 1)
                .reshape(-1))
    table_rm = _make_tc_transpose(v)(table.T).reshape(v * 8, d)
    out5 = _make_kernel(v)(idx_phys, table_rm)
    return out5.transpose(2, 4, 0, 1, 3).reshape(_B, _H, _D)
